# Initial kernel scaffold; baseline (speedup 1.0000x reference)
import jax
import jax.numpy as jnp
from jax.experimental import pallas as pl


def kernel(label_cls, label_loc, label_loc_weight, rpn_pred_cls, rpn_pred_loc):
    raise NotImplementedError("write your pallas kernel here")


# speedup vs baseline: 3.4167x; 3.4167x over previous
import functools

import jax
import jax.numpy as jnp
from jax import lax
from jax.experimental import pallas as pl
from jax.experimental.pallas import tpu as pltpu
from jax.experimental.pallas import tpu_sc as plsc

NC, NS, L = 2, 16, 16
NW = NC * NS
B = 128
NLAB = 5 * 25 * 25
NPC = 2 * NLAB
NLOC = 4 * NLAB
BPW = B // NW
NCHUNK = (NLAB + L - 1) // L
NTOT = B * NLAB


def _sc_body(lab_hbm, pc_hbm, plc_hbm, ll_hbm, w_hbm, out_hbm,
             lab_v0, pc_v0, plc_v0, ll_v0, w_v0,
             lab_v1, pc_v1, plc_v1, ll_v1, w_v1,
             st_v, sem0, sem1):
    wid = lax.axis_index("s") * NC + lax.axis_index("c")
    bufs = ((lab_v0, pc_v0, plc_v0, ll_v0, w_v0),
            (lab_v1, pc_v1, plc_v1, ll_v1, w_v1))
    sems = (sem0, sem1)

    def start(i):
        b = wid * BPW + i
        lab_v, pc_v, plc_v, ll_v, w_v = bufs[i % 2]
        sem = sems[i % 2]
        return (pltpu.async_copy(lab_hbm.at[b], lab_v, sem),
                pltpu.async_copy(pc_hbm.at[b], pc_v, sem),
                pltpu.async_copy(plc_hbm.at[b], plc_v, sem),
                pltpu.async_copy(ll_hbm.at[b], ll_v, sem),
                pltpu.async_copy(w_hbm.at[b], w_v, sem))

    iota = lax.iota(jnp.int32, L)
    zeros = jnp.zeros((L,), jnp.float32)
    ones = jnp.ones((L,), jnp.float32)
    acc = (zeros, zeros, zeros, zeros)

    handles = start(0)
    for i in range(BPW):
        for h in handles:
            h.wait()
        if i + 1 < BPW:
            handles = start(i + 1)
        lab_v, pc_v, plc_v, ll_v, w_v = bufs[i % 2]

        def body(t, carry, lab_v=lab_v, pc_v=pc_v, plc_v=plc_v,
                 ll_v=ll_v, w_v=w_v):
            a_pos, a_neg, n_pos, a_loc = carry
            j0 = t * L
            lane = j0 + iota
            mask = lane < NLAB
            lab = lab_v[pl.ds(j0, L)]
            lab = jnp.where(mask, lab, -2)
            idx = jnp.where(mask, 2 * lane + lab, 0)
            logp = plsc.load_gather(pc_v, [idx])
            pos = lab == 1
            neg = lab == 0
            a_pos = a_pos + jnp.where(pos, logp, zeros)
            a_neg = a_neg + jnp.where(neg, logp, zeros)
            n_pos = n_pos + jnp.where(pos, ones, zeros)
            w = w_v[pl.ds(j0, L)]
            d = jnp.abs(plc_v[pl.ds(j0, L)] - ll_v[pl.ds(j0, L)])
            for c in range(1, 4):
                d = d + jnp.abs(plc_v[pl.ds(c * NLAB + j0, L)]
                                - ll_v[pl.ds(c * NLAB + j0, L)])
            a_loc = a_loc + jnp.where(mask, d * w, zeros)
            return a_pos, a_neg, n_pos, a_loc

        acc = lax.fori_loop(0, NCHUNK, body, acc)

    for c in range(4):
        st_v[c] = acc[c]
    pltpu.sync_copy(st_v, out_hbm.at[wid])


_sc_partials = functools.partial(
    pl.kernel,
    out_type=jax.ShapeDtypeStruct((NW, 4, L), jnp.float32),
    mesh=plsc.VectorSubcoreMesh(core_axis_name="c", subcore_axis_name="s"),
    compiler_params=pltpu.CompilerParams(needs_layout_passes=False),
    scratch_types=[
        pltpu.VMEM((NLAB,), jnp.int32),
        pltpu.VMEM((NPC,), jnp.float32),
        pltpu.VMEM((NLOC,), jnp.float32),
        pltpu.VMEM((NLOC,), jnp.float32),
        pltpu.VMEM((NLAB,), jnp.float32),
        pltpu.VMEM((NLAB,), jnp.int32),
        pltpu.VMEM((NPC,), jnp.float32),
        pltpu.VMEM((NLOC,), jnp.float32),
        pltpu.VMEM((NLOC,), jnp.float32),
        pltpu.VMEM((NLAB,), jnp.float32),
        pltpu.VMEM((4, L), jnp.float32),
        pltpu.SemaphoreType.DMA,
        pltpu.SemaphoreType.DMA,
    ],
)(_sc_body)


def _combine_body(p_ref, o_ref):
    p = p_ref[...]
    a_pos = jnp.sum(p[:, 0, :])
    a_neg = jnp.sum(p[:, 1, :])
    n_pos = jnp.sum(p[:, 2, :])
    a_loc = jnp.sum(p[:, 3, :])
    n_neg = float(NTOT) - n_pos
    loss_cls = 0.5 * (-a_pos / jnp.maximum(n_pos, 1.0)
                      - a_neg / jnp.maximum(n_neg, 1.0))
    o_ref[0, 0] = loss_cls + 1.2 * (a_loc / float(B))


_combine = pl.pallas_call(
    _combine_body,
    out_shape=jax.ShapeDtypeStruct((1, 1), jnp.float32),
    out_specs=pl.BlockSpec(memory_space=pltpu.SMEM),
)


def kernel(label_cls, label_loc, label_loc_weight, rpn_pred_cls, rpn_pred_loc):
    lab = label_cls.reshape(B, NLAB).astype(jnp.int32)
    pc = rpn_pred_cls.reshape(B, NPC)
    plc = rpn_pred_loc.reshape(B, NLOC)
    ll = label_loc.reshape(B, NLOC)
    w = label_loc_weight.reshape(B, NLAB)
    partials = _sc_partials(lab, pc, plc, ll, w)
    return _combine(partials)[0, 0]
